# baseline (speedup 1.0000x reference)
import functools

import jax
import jax.numpy as jnp
from jax import lax
from jax.experimental import pallas as pl
from jax.experimental.pallas import tpu as pltpu
from jax.experimental.pallas import tpu_sc as plsc

VOCAB = 1000000
DIM = 64
OUT = 4
SEQ = 200
BATCH = 4096
LANES = 16
NC, NS = 2, 16
NW = NC * NS
BPW = BATCH // NW

CHUNK = 20
NCHUNK = SEQ // CHUNK

VBLK = 32768
GRID = -(-VOCAB // VBLK)
OBLK = VBLK // 8
OSHIFT = OBLK.bit_length() - 1
NBLK_A = 16
HALF = NBLK_A * VBLK
NBLK_B = 1 + (GRID - NBLK_A)


def _project_body(tT_ref, w_ref, o_ref):
    tT = tT_ref[...].astype(jnp.bfloat16)
    w = (w_ref[...] * (1.0 / SEQ)).astype(jnp.bfloat16)
    packed = lax.dot_general(
        tT, w, (((0,), (1,)), ((), ())),
        preferred_element_type=jnp.float32)
    for s in range(8):
        o_ref[:, s * LANES:(s + 1) * LANES] = packed[s * OBLK:(s + 1) * OBLK]


def _project_part(tableT, w_pad, imap, nblocks):
    return pl.pallas_call(
        _project_body,
        grid=(nblocks,),
        in_specs=[
            pl.BlockSpec((DIM, VBLK), lambda i: (0, imap(i))),
            pl.BlockSpec((LANES, DIM), lambda i: (0, 0)),
        ],
        out_specs=pl.BlockSpec((OBLK, 128), lambda i: (i, 0)),
        out_shape=jax.ShapeDtypeStruct((nblocks * OBLK, 128), jnp.float32),
        compiler_params=pltpu.CompilerParams(
            dimension_semantics=("parallel",),
            fuse_transposed_lhs_in_matmul=True,
        ),
    )(tableT, w_pad)


def _permute_body(t_ref, a_ref, b_ref):
    v = t_ref[...]
    pi = (
        jnp.bitwise_and(v, ~(VBLK - 1))
        | jnp.left_shift(jnp.bitwise_and(v, OBLK - 1), 3)
        | jnp.right_shift(jnp.bitwise_and(v, VBLK - 1), OSHIFT)
    )
    in_a = pi < HALF
    a_ref[...] = jnp.where(in_a, pi, 0)
    b_ref[...] = jnp.where(in_a, 0, pi - HALF + VBLK)


def _permute_text(text):
    return pl.pallas_call(
        _permute_body,
        grid=(1,),
        in_specs=[pl.BlockSpec((SEQ, BATCH), lambda i: (0, 0))],
        out_specs=[pl.BlockSpec((SEQ, BATCH), lambda i: (0, 0))] * 2,
        out_shape=[jax.ShapeDtypeStruct((SEQ, BATCH), jnp.int32)] * 2,
    )(text)


def _gather_sum(text, proj, acc_init):
    mesh = plsc.VectorSubcoreMesh(core_axis_name="c", subcore_axis_name="s")

    @functools.partial(
        pl.kernel,
        out_type=jax.ShapeDtypeStruct((BATCH, LANES), jnp.float32),
        mesh=mesh,
        scratch_types=[
            pltpu.VMEM((SEQ, BPW), jnp.int32),
            pltpu.VMEM((2, CHUNK, BPW, LANES), jnp.float32),
            pltpu.VMEM((BPW, LANES), jnp.float32),
            pltpu.SemaphoreType.DMA,
            pltpu.SemaphoreType.DMA,
        ],
        compiler_params=pltpu.CompilerParams(use_tc_tiling_on_sc=False),
    )
    def k(text_hbm, p_hbm, init_hbm, out_hbm, idx_v, rows_v, acc_v,
          sem0, sem1):
        wid = lax.axis_index("s") * NC + lax.axis_index("c")
        base = wid * BPW
        pltpu.sync_copy(text_hbm.at[:, pl.ds(base, BPW)], idx_v)
        pltpu.sync_copy(init_hbm.at[pl.ds(base, BPW)], acc_v)

        def gather(c, buf, sem):
            for r in range(CHUNK):
                pltpu.async_copy(
                    p_hbm.at[idx_v.at[c * CHUNK + r]],
                    rows_v.at[buf, r], sem)

        def wait(c, buf, sem):
            for r in range(CHUNK):
                pltpu.make_async_copy(
                    p_hbm.at[idx_v.at[c * CHUNK + r]],
                    rows_v.at[buf, r], sem).wait()

        def accumulate(buf):
            @pl.loop(0, BPW)
            def _(i):
                acc = acc_v[i, :]
                for r in range(CHUNK):
                    acc = acc + rows_v[buf, r, i, :]
                acc_v[i, :] = acc

        gather(0, 0, sem0)

        @pl.loop(0, NCHUNK, step=2)
        def _(c):
            gather(c + 1, 1, sem1)
            wait(c, 0, sem0)
            accumulate(0)

            @pl.when(c + 2 < NCHUNK)
            def _():
                gather(c + 2, 0, sem0)

            wait(c + 1, 1, sem1)
            accumulate(1)

        pltpu.sync_copy(acc_v, out_hbm.at[pl.ds(base, BPW)])

    return k(text, proj, acc_init)


def kernel(text, table, W, b):
    w_pad = jnp.zeros((LANES, DIM), jnp.float32).at[:OUT].set(W)
    b_pad = jnp.zeros((LANES,), jnp.float32).at[:OUT].set(b)
    tableT = table.T
    proj_a = _project_part(
        tableT, w_pad, lambda i: i, NBLK_A).reshape(HALF, LANES)
    proj_b = _project_part(
        tableT, w_pad, lambda i: jnp.where(i == 0, 0, i + NBLK_A - 1),
        NBLK_B).reshape(NBLK_B * VBLK, LANES)
    text_a, text_b = _permute_text(text)
    init = jnp.broadcast_to(b_pad, (BATCH, LANES))
    sums_a = _gather_sum(text_a, proj_a, init)
    sums = _gather_sum(text_b, proj_b, sums_a)
    return sums[:, :OUT]


# speedup vs baseline: 14.6748x; 14.6748x over previous
import functools

import jax
import jax.numpy as jnp
from jax import lax
from jax.experimental import pallas as pl
from jax.experimental.pallas import tpu as pltpu
from jax.experimental.pallas import tpu_sc as plsc

VOCAB = 1000000
DIM = 64
OUT = 4
SEQ = 200
BATCH = 4096
LANES = 16
NC, NS = 2, 16
NW = NC * NS
BPW = BATCH // NW

CHUNK = 20
NCHUNK = SEQ // CHUNK

VBLK = 32768
GRID = -(-VOCAB // VBLK)
OBLK = VBLK // 8
OSHIFT = OBLK.bit_length() - 1
NBLK_A = 16
HALF = NBLK_A * VBLK
NBLK_B = GRID - NBLK_A


def _make_project_body(imap):
    def body(tT_ref, w_ref, o_ref):
        tT = tT_ref[...].astype(jnp.bfloat16)
        w = (w_ref[...] * (1.0 / SEQ)).astype(jnp.bfloat16)
        packed = lax.dot_general(
            tT, w, (((0,), (1,)), ((), ())),
            preferred_element_type=jnp.float32)
        base = imap(pl.program_id(0)) * VBLK
        valid = (lax.broadcasted_iota(jnp.int32, (VBLK, 1), 0) + base
                 ) < VOCAB
        packed = jnp.where(valid, packed, 0.0)
        for s in range(8):
            o_ref[:, s * LANES:(s + 1) * LANES] = \
                packed[s * OBLK:(s + 1) * OBLK]
    return body


def _project_part(tableT, w_pad, imap, nblocks):
    return pl.pallas_call(
        _make_project_body(imap),
        grid=(nblocks,),
        in_specs=[
            pl.BlockSpec((DIM, VBLK), lambda i: (0, imap(i))),
            pl.BlockSpec((LANES, DIM), lambda i: (0, 0)),
        ],
        out_specs=pl.BlockSpec((OBLK, 128), lambda i: (i, 0)),
        out_shape=jax.ShapeDtypeStruct((nblocks * OBLK, 128), jnp.float32),
        compiler_params=pltpu.CompilerParams(
            dimension_semantics=("parallel",),
            fuse_transposed_lhs_in_matmul=True,
        ),
    )(tableT, w_pad)


def _permute_body(t_ref, a_ref, b_ref):
    v = t_ref[...]
    pi = (
        jnp.bitwise_and(v, ~(VBLK - 1))
        | jnp.left_shift(jnp.bitwise_and(v, OBLK - 1), 3)
        | jnp.right_shift(jnp.bitwise_and(v, VBLK - 1), OSHIFT)
    )
    in_a = pi < HALF
    zoff = (jnp.left_shift(jnp.bitwise_and(v, OBLK - 1), 3) + 6
            + jnp.bitwise_and(jnp.right_shift(v, OSHIFT), 1))
    a_ref[...] = jnp.where(in_a, pi, NBLK_A * VBLK + zoff)
    b_ref[...] = jnp.where(in_a, (NBLK_B - 1) * VBLK + zoff, pi - HALF)


def _permute_text(text):
    return pl.pallas_call(
        _permute_body,
        grid=(1,),
        in_specs=[pl.BlockSpec((SEQ, BATCH), lambda i: (0, 0))],
        out_specs=[pl.BlockSpec((SEQ, BATCH), lambda i: (0, 0))] * 2,
        out_shape=[jax.ShapeDtypeStruct((SEQ, BATCH), jnp.int32)] * 2,
    )(text)


def _gather_sum(text, proj, acc_init):
    mesh = plsc.VectorSubcoreMesh(core_axis_name="c", subcore_axis_name="s")

    @functools.partial(
        pl.kernel,
        out_type=jax.ShapeDtypeStruct((BATCH, LANES), jnp.float32),
        mesh=mesh,
        scratch_types=[
            pltpu.VMEM((SEQ, BPW), jnp.int32),
            pltpu.VMEM((2, CHUNK, BPW, LANES), jnp.float32),
            pltpu.VMEM((BPW, LANES), jnp.float32),
            pltpu.SemaphoreType.DMA,
            pltpu.SemaphoreType.DMA,
        ],
        compiler_params=pltpu.CompilerParams(use_tc_tiling_on_sc=False),
    )
    def k(text_hbm, p_hbm, init_hbm, out_hbm, idx_v, rows_v, acc_v,
          sem0, sem1):
        wid = lax.axis_index("s") * NC + lax.axis_index("c")
        base = wid * BPW
        pltpu.sync_copy(text_hbm.at[:, pl.ds(base, BPW)], idx_v)
        pltpu.sync_copy(init_hbm.at[pl.ds(base, BPW)], acc_v)

        def gather(c, buf, sem):
            for r in range(CHUNK):
                pltpu.async_copy(
                    p_hbm.at[idx_v.at[c * CHUNK + r]],
                    rows_v.at[buf, r], sem)

        def wait(c, buf, sem):
            for r in range(CHUNK):
                pltpu.make_async_copy(
                    p_hbm.at[idx_v.at[c * CHUNK + r]],
                    rows_v.at[buf, r], sem).wait()

        def accumulate(buf):
            @pl.loop(0, BPW)
            def _(i):
                acc = acc_v[i, :]
                for r in range(CHUNK):
                    acc = acc + rows_v[buf, r, i, :]
                acc_v[i, :] = acc

        gather(0, 0, sem0)

        @pl.loop(0, NCHUNK, step=2)
        def _(c):
            gather(c + 1, 1, sem1)
            wait(c, 0, sem0)
            accumulate(0)

            @pl.when(c + 2 < NCHUNK)
            def _():
                gather(c + 2, 0, sem0)

            wait(c + 1, 1, sem1)
            accumulate(1)

        pltpu.sync_copy(acc_v, out_hbm.at[pl.ds(base, BPW)])

    return k(text, proj, acc_init)


def kernel(text, table, W, b):
    w_pad = jnp.zeros((LANES, DIM), jnp.float32).at[:OUT].set(W)
    b_pad = jnp.zeros((LANES,), jnp.float32).at[:OUT].set(b)
    tableT = table.T
    proj_a = _project_part(
        tableT, w_pad, lambda i: jnp.where(i == NBLK_A, GRID - 1, i),
        NBLK_A + 1).reshape((NBLK_A + 1) * VBLK, LANES)
    proj_b = _project_part(
        tableT, w_pad, lambda i: i + NBLK_A,
        NBLK_B).reshape(NBLK_B * VBLK, LANES)
    text_a, text_b = _permute_text(text)
    init = jnp.broadcast_to(b_pad, (BATCH, LANES))
    sums_a = _gather_sum(text_a, proj_a, init)
    sums = _gather_sum(text_b, proj_b, sums_a)
    return sums[:, :OUT]


# speedup vs baseline: 15.4546x; 1.0531x over previous
import functools

import jax
import jax.numpy as jnp
from jax import lax
from jax.experimental import pallas as pl
from jax.experimental.pallas import tpu as pltpu
from jax.experimental.pallas import tpu_sc as plsc

VOCAB = 1000000
DIM = 64
OUT = 4
SEQ = 200
BATCH = 4096
LANES = 16
NC, NS = 2, 16
NW = NC * NS
BPW = BATCH // NW

CHUNK = 20
NCHUNK = SEQ // CHUNK

VBLK = 32768
GRID = -(-VOCAB // VBLK)
VPAD = GRID * VBLK
OBLK = VBLK // 8
OSHIFT = OBLK.bit_length() - 1


def _project_body(tT_ref, w_ref, o_ref):
    tT = tT_ref[...].astype(jnp.bfloat16)
    w = (w_ref[...] * (1.0 / SEQ)).astype(jnp.bfloat16)
    packed = lax.dot_general(
        tT, w, (((0,), (1,)), ((), ())),
        preferred_element_type=jnp.float32)
    for s in range(8):
        o_ref[:, s * LANES:(s + 1) * LANES] = packed[s * OBLK:(s + 1) * OBLK]


def _project(tableT, w_pad):
    return pl.pallas_call(
        _project_body,
        grid=(GRID,),
        in_specs=[
            pl.BlockSpec((DIM, VBLK), lambda i: (0, i)),
            pl.BlockSpec((LANES, DIM), lambda i: (0, 0)),
        ],
        out_specs=pl.BlockSpec((OBLK, 128), lambda i: (i, 0)),
        out_shape=jax.ShapeDtypeStruct((VPAD // 8, 128), jnp.float32),
        compiler_params=pltpu.CompilerParams(
            dimension_semantics=("parallel",),
            fuse_transposed_lhs_in_matmul=True,
        ),
    )(tableT, w_pad)


def _permute_body(t_ref, o_ref):
    v = t_ref[...]
    o_ref[...] = (
        jnp.bitwise_and(v, ~(VBLK - 1))
        | jnp.left_shift(jnp.bitwise_and(v, OBLK - 1), 3)
        | jnp.right_shift(jnp.bitwise_and(v, VBLK - 1), OSHIFT)
    )


def _permute_text(text):
    return pl.pallas_call(
        _permute_body,
        grid=(1,),
        in_specs=[pl.BlockSpec((SEQ, BATCH), lambda i: (0, 0))],
        out_specs=pl.BlockSpec((SEQ, BATCH), lambda i: (0, 0)),
        out_shape=jax.ShapeDtypeStruct((SEQ, BATCH), jnp.int32),
    )(text)


def _gather_sum(text, proj, bias_pad):
    mesh = plsc.VectorSubcoreMesh(core_axis_name="c", subcore_axis_name="s")

    @functools.partial(
        pl.kernel,
        out_type=jax.ShapeDtypeStruct((BATCH, LANES), jnp.float32),
        mesh=mesh,
        scratch_types=[
            pltpu.VMEM((SEQ, BPW), jnp.int32),
            pltpu.VMEM((2, CHUNK, BPW, LANES), jnp.float32),
            pltpu.VMEM((BPW, LANES), jnp.float32),
            pltpu.VMEM((LANES,), jnp.float32),
            pltpu.SemaphoreType.DMA,
            pltpu.SemaphoreType.DMA,
        ],
        compiler_params=pltpu.CompilerParams(use_tc_tiling_on_sc=False),
    )
    def k(text_hbm, p_hbm, b_hbm, out_hbm, idx_v, rows_v, acc_v, b_v,
          sem0, sem1):
        wid = lax.axis_index("s") * NC + lax.axis_index("c")
        base = wid * BPW
        pltpu.sync_copy(text_hbm.at[:, pl.ds(base, BPW)], idx_v)
        pltpu.sync_copy(b_hbm, b_v)
        bias = b_v[...]

        @pl.loop(0, BPW)
        def _(i):
            acc_v[i, :] = bias

        def gather(c, buf, sem):
            for r in range(CHUNK):
                pltpu.async_copy(
                    p_hbm.at[idx_v.at[c * CHUNK + r]],
                    rows_v.at[buf, r], sem)

        def wait(c, buf, sem):
            for r in range(CHUNK):
                pltpu.make_async_copy(
                    p_hbm.at[idx_v.at[c * CHUNK + r]],
                    rows_v.at[buf, r], sem).wait()

        def accumulate(buf):
            @pl.loop(0, BPW)
            def _(i):
                acc = acc_v[i, :]
                for r in range(CHUNK):
                    acc = acc + rows_v[buf, r, i, :]
                acc_v[i, :] = acc

        gather(0, 0, sem0)

        @pl.loop(0, NCHUNK, step=2)
        def _(c):
            gather(c + 1, 1, sem1)
            wait(c, 0, sem0)
            accumulate(0)

            @pl.when(c + 2 < NCHUNK)
            def _():
                gather(c + 2, 0, sem0)

            wait(c + 1, 1, sem1)
            accumulate(1)

        pltpu.sync_copy(acc_v, out_hbm.at[pl.ds(base, BPW)])

    return k(text, proj, bias_pad)


def kernel(text, table, W, b):
    w_pad = jnp.zeros((LANES, DIM), jnp.float32).at[:OUT].set(W)
    b_pad = jnp.zeros((LANES,), jnp.float32).at[:OUT].set(b)
    proj = _project(table.T, w_pad).reshape(VPAD, LANES)
    sums = _gather_sum(_permute_text(text), proj, b_pad)
    return sums[:, :OUT]
